# Initial kernel scaffold; baseline (speedup 1.0000x reference)
import jax
import jax.numpy as jnp
from jax.experimental import pallas as pl


def kernel(x):
    raise NotImplementedError("write your pallas kernel here")


# speedup vs baseline: 1.4281x; 1.4281x over previous
import functools

import jax
import jax.numpy as jnp
from jax import lax
from jax.experimental import pallas as pl
from jax.experimental.pallas import tpu as pltpu
from jax.experimental.pallas import tpu_sc as plsc

_X_MIN = -5.0
_X_MAX = 5.0
_NBINS = 256

_NC = 2
_NS = 16
_NW = _NC * _NS
_LANES = 16

_TOTAL = 4096 * 8192
_PER_W = _TOTAL // _NW
_CHUNK = 65536
_NCHUNKS = _PER_W // _CHUNK


def _hist_body(x_hbm, out_hbm, buf, hist, lhist):
    cid = lax.axis_index("c")
    sid = lax.axis_index("s")
    wid = sid * _NC + cid
    base = wid * _PER_W

    zeros16 = jnp.zeros((_LANES,), jnp.int32)

    def zero_seg(t, _):
        hist[pl.ds(t * _LANES, _LANES)] = zeros16
        return 0

    lax.fori_loop(0, _NS * _NBINS // _LANES, zero_seg, 0)

    lane_base = lax.iota(jnp.int32, _LANES) * _NBINS
    ones = jnp.ones((_LANES,), jnp.int32)
    scale = jnp.float32((_NBINS) / (_X_MAX - _X_MIN))
    shift = jnp.float32(-_X_MIN * _NBINS / (_X_MAX - _X_MIN))
    hi = jnp.float32(_NBINS - 1)

    def chunk_body(ci, _):
        pltpu.sync_copy(x_hbm.at[pl.ds(base + ci * _CHUNK, _CHUNK)], buf)

        def vec_body(j, _):
            v = buf[pl.ds(j * _LANES, _LANES)]
            t = v * scale + shift
            t = jnp.minimum(jnp.maximum(t, 0.0), hi)
            idx = t.astype(jnp.int32) + lane_base
            plsc.addupdate_scatter(hist, [idx], ones)
            return 0

        lax.fori_loop(0, _CHUNK // _LANES, vec_body, 0)
        return 0

    lax.fori_loop(0, _NCHUNKS, chunk_body, 0)

    def red_seg(t, _):
        acc = hist[pl.ds(t * _LANES, _LANES)]
        for r in range(1, _NS):
            acc = acc + hist[pl.ds(r * _NBINS + t * _LANES, _LANES)]
        lhist[pl.ds(t * _LANES, _LANES)] = acc
        return 0

    lax.fori_loop(0, _NBINS // _LANES, red_seg, 0)
    pltpu.sync_copy(lhist, out_hbm.at[wid])


@functools.partial(jax.jit)
def _histogram(x_flat):
    mesh = plsc.VectorSubcoreMesh(
        core_axis_name="c", subcore_axis_name="s",
        num_cores=_NC, num_subcores=_NS)
    partials = pl.kernel(
        _hist_body,
        out_type=jax.ShapeDtypeStruct((_NW, _NBINS), jnp.int32),
        mesh=mesh,
        compiler_params=pltpu.CompilerParams(needs_layout_passes=False),
        scratch_types=[
            pltpu.VMEM((_CHUNK,), jnp.float32),
            pltpu.VMEM((_NS * _NBINS,), jnp.int32),
            pltpu.VMEM((_NBINS,), jnp.int32),
        ],
    )(x_flat)
    return jnp.sum(partials, axis=0)


def kernel(x):
    hist = _histogram(x.reshape(-1)).astype(jnp.int64)
    return (x, hist)


# speedup vs baseline: 4.3269x; 3.0298x over previous
import functools

import jax
import jax.numpy as jnp
from jax import lax
from jax.experimental import pallas as pl
from jax.experimental.pallas import tpu as pltpu
from jax.experimental.pallas import tpu_sc as plsc

_X_MIN = -5.0
_X_MAX = 5.0
_NBINS = 256

_NC = 2
_NS = 16
_NW = _NC * _NS
_LANES = 16

_TOTAL = 4096 * 8192
_PER_W = _TOTAL // _NW
_CHUNK = 32768
_NCHUNKS = _PER_W // _CHUNK
_UNROLL = 8


def _hist_body(x_hbm, out_hbm, buf0, buf1, hist, lhist, sem0, sem1):
    cid = lax.axis_index("c")
    sid = lax.axis_index("s")
    wid = sid * _NC + cid
    base = wid * _PER_W

    zeros16 = jnp.zeros((_LANES,), jnp.int32)

    def zero_seg(t, _):
        hist[pl.ds(t * _LANES, _LANES)] = zeros16
        return 0

    lax.fori_loop(0, _NS * _NBINS // _LANES, zero_seg, 0)

    lane_base = (lax.iota(jnp.int32, _LANES) * _NBINS).astype(jnp.float32)
    ones = jnp.ones((_LANES,), jnp.int32)
    scale = jnp.float32(_NBINS / (_X_MAX - _X_MIN))
    shiftv = lane_base + jnp.float32(-_X_MIN * _NBINS / (_X_MAX - _X_MIN))
    lov = lane_base
    hiv = lane_base + jnp.float32(_NBINS - 1)

    def copy_in(ci, bref, sem):
        return pltpu.make_async_copy(
            x_hbm.at[pl.ds(base + ci * _CHUNK, _CHUNK)], bref, sem)

    def compute(bref):
        def vec_body(j, _):
            b = j * (_LANES * _UNROLL)
            vs = [bref[pl.ds(b + u * _LANES, _LANES)]
                  for u in range(_UNROLL)]
            idxs = []
            for v in vs:
                t = v * scale + shiftv
                t = jnp.minimum(jnp.maximum(t, lov), hiv)
                idxs.append(t.astype(jnp.int32))
            for ix in idxs:
                plsc.addupdate_scatter(hist, [ix], ones)
            return 0

        lax.fori_loop(0, _CHUNK // (_LANES * _UNROLL), vec_body, 0)

    copy_in(0, buf0, sem0).start()
    copy_in(1, buf1, sem1).start()

    def pair_body(p, _):
        c = 2 * p
        copy_in(c, buf0, sem0).wait()
        compute(buf0)

        @pl.when(c + 2 < _NCHUNKS)
        def _():
            copy_in(c + 2, buf0, sem0).start()

        copy_in(c + 1, buf1, sem1).wait()
        compute(buf1)

        @pl.when(c + 3 < _NCHUNKS)
        def _():
            copy_in(c + 3, buf1, sem1).start()

        return 0

    lax.fori_loop(0, _NCHUNKS // 2, pair_body, 0)

    def red_seg(t, _):
        acc = hist[pl.ds(t * _LANES, _LANES)]
        for r in range(1, _NS):
            acc = acc + hist[pl.ds(r * _NBINS + t * _LANES, _LANES)]
        lhist[pl.ds(t * _LANES, _LANES)] = acc
        return 0

    lax.fori_loop(0, _NBINS // _LANES, red_seg, 0)
    pltpu.sync_copy(lhist, out_hbm.at[wid])


@functools.partial(jax.jit)
def _histogram(x_flat):
    mesh = plsc.VectorSubcoreMesh(
        core_axis_name="c", subcore_axis_name="s",
        num_cores=_NC, num_subcores=_NS)
    partials = pl.kernel(
        _hist_body,
        out_type=jax.ShapeDtypeStruct((_NW, _NBINS), jnp.int32),
        mesh=mesh,
        compiler_params=pltpu.CompilerParams(needs_layout_passes=False),
        scratch_types=[
            pltpu.VMEM((_CHUNK,), jnp.float32),
            pltpu.VMEM((_CHUNK,), jnp.float32),
            pltpu.VMEM((_NS * _NBINS,), jnp.int32),
            pltpu.VMEM((_NBINS,), jnp.int32),
            pltpu.SemaphoreType.DMA,
            pltpu.SemaphoreType.DMA,
        ],
    )(x_flat)
    return jnp.sum(partials, axis=0)


def kernel(x):
    hist = _histogram(x.reshape(-1)).astype(jnp.int64)
    return (x, hist)
